# baseline (speedup 1.0000x reference)
import functools

import jax
import jax.numpy as jnp
from jax import lax
from jax.experimental import pallas as pl
from jax.experimental.pallas import tpu as pltpu
from jax.experimental.pallas import tpu_sc as plsc

N_CLS = 1000
ROWS = 1024 * 26
NC, NS, L = 2, 16, 16
NW = NC * NS
RPW = ROWS // NW
CH = 32
NCH = RPW // CH
GRP = CH // L

_mesh = plsc.VectorSubcoreMesh(
    core_axis_name="c", subcore_axis_name="s", num_cores=NC, num_subcores=NS
)


@functools.partial(
    pl.kernel,
    out_type=jax.ShapeDtypeStruct((ROWS * N_CLS,), jnp.int32),
    mesh=_mesh,
    compiler_params=pltpu.CompilerParams(needs_layout_passes=False),
    scratch_types=[
        pltpu.VMEM((RPW,), jnp.int32),
        pltpu.VMEM((CH * N_CLS,), jnp.int32),
        pltpu.VMEM((CH * N_CLS,), jnp.int32),
        pltpu.SemaphoreType.DMA,
        pltpu.SemaphoreType.DMA,
    ],
)
def _onehot_sc(x_hbm, zeros_hbm, out_hbm, idx_v, buf0, buf1, sem0, sem1):
    wid = lax.axis_index("s") * NC + lax.axis_index("c")
    row0 = wid * RPW

    pltpu.sync_copy(x_hbm.at[pl.ds(row0 * 1, RPW)], idx_v)
    pltpu.sync_copy(zeros_hbm, buf0)
    pltpu.sync_copy(zeros_hbm, buf1)

    bufs = (buf0, buf1)
    sems = (sem0, sem1)
    descs = [None, None]
    lane_rows = lax.iota(jnp.int32, L) * N_CLS
    ones = jnp.ones((L,), jnp.int32)
    zeros = jnp.zeros((L,), jnp.int32)

    def flat_pos(c, g):
        xv = idx_v[pl.ds(c * CH + g * L, L)]
        return xv + lane_rows + g * (L * N_CLS)

    for c in range(NCH):
        b = c % 2
        if c >= 2:
            descs[b].wait()
            for g in range(GRP):
                plsc.store_scatter(bufs[b], [flat_pos(c - 2, g)], zeros)
        for g in range(GRP):
            plsc.store_scatter(bufs[b], [flat_pos(c, g)], ones)
        off = (row0 + c * CH) * N_CLS
        descs[b] = pltpu.async_copy(
            bufs[b], out_hbm.at[pl.ds(off, CH * N_CLS)], sems[b]
        )
    descs[0].wait()
    descs[1].wait()


def kernel(x):
    xf = x.reshape(-1).astype(jnp.int32)
    z = jnp.zeros((CH * N_CLS,), jnp.int32)
    out = _onehot_sc(xf, z)
    return out.reshape(x.shape[0], x.shape[1], N_CLS)
